# baseline (device time: 16558 ns/iter reference)
import jax
import jax.numpy as jnp
from jax import lax
from jax.experimental import pallas as pl
from jax.experimental.pallas import tpu as pltpu

N_DEV = 8


def kernel(x, w_mat):
    m, k_per = x.shape
    _, n = w_mat.shape
    m_out = m // N_DEV

    def body(x_ref, w_ref, out_ref, sq_ref, ss_ref, rq_ref, rs_ref,
             qsend_sems, qrecv_sems, ssend_sems, srecv_sems):
        p = lax.axis_index("i")

        barrier_sem = pltpu.get_barrier_semaphore()
        for k in range(1, N_DEV):
            peer = lax.rem(p + k, N_DEV)
            pl.semaphore_signal(
                barrier_sem, inc=1,
                device_id=(peer,), device_id_type=pl.DeviceIdType.MESH,
            )
        pl.semaphore_wait(barrier_sem, N_DEV - 1)

        def stage_and_send(k, carry):
            peer = lax.rem(p + k, N_DEV)
            xk = x_ref[pl.ds(peer * m_out, m_out), :]
            c = jnp.dot(xk, w_ref[:, :], preferred_element_type=jnp.float32)
            scale = jnp.maximum(jnp.max(jnp.abs(c)), 1e-30) / 127.0
            q = jnp.clip(jnp.round(c / scale), -127.0, 127.0).astype(jnp.int8)
            sq_ref[pl.ds(k - 1, 1)] = q[None]
            ss_ref[pl.ds(k - 1, 1)] = jnp.full((1, 8, 128), scale, jnp.float32)

            data = pltpu.make_async_remote_copy(
                src_ref=sq_ref.at[k - 1],
                dst_ref=rq_ref.at[k - 1],
                send_sem=qsend_sems.at[k - 1],
                recv_sem=qrecv_sems.at[k - 1],
                device_id=(peer,),
                device_id_type=pl.DeviceIdType.MESH,
            )
            data.start()
            sc = pltpu.make_async_remote_copy(
                src_ref=ss_ref.at[k - 1],
                dst_ref=rs_ref.at[k - 1],
                send_sem=ssend_sems.at[k - 1],
                recv_sem=srecv_sems.at[k - 1],
                device_id=(peer,),
                device_id_type=pl.DeviceIdType.MESH,
            )
            sc.start()
            return carry

        lax.fori_loop(1, N_DEV, stage_and_send, 0)

        total = jnp.dot(
            x_ref[pl.ds(p * m_out, m_out), :], w_ref[:, :],
            preferred_element_type=jnp.float32,
        )

        for k in range(1, N_DEV):
            data = pltpu.make_async_remote_copy(
                src_ref=sq_ref.at[k - 1],
                dst_ref=rq_ref.at[k - 1],
                send_sem=qsend_sems.at[k - 1],
                recv_sem=qrecv_sems.at[k - 1],
                device_id=(p,),
                device_id_type=pl.DeviceIdType.MESH,
            )
            sc = pltpu.make_async_remote_copy(
                src_ref=ss_ref.at[k - 1],
                dst_ref=rs_ref.at[k - 1],
                send_sem=ssend_sems.at[k - 1],
                recv_sem=srecv_sems.at[k - 1],
                device_id=(p,),
                device_id_type=pl.DeviceIdType.MESH,
            )
            data.wait_recv()
            sc.wait_recv()
            scale = rs_ref[k - 1, 0:1, 0:1]
            total = total + rq_ref[k - 1].astype(jnp.float32) * scale

        out_ref[:, :] = total * jax.nn.sigmoid(total)

        for k in range(1, N_DEV):
            data = pltpu.make_async_remote_copy(
                src_ref=sq_ref.at[k - 1],
                dst_ref=rq_ref.at[k - 1],
                send_sem=qsend_sems.at[k - 1],
                recv_sem=qrecv_sems.at[k - 1],
                device_id=(p,),
                device_id_type=pl.DeviceIdType.MESH,
            )
            sc = pltpu.make_async_remote_copy(
                src_ref=ss_ref.at[k - 1],
                dst_ref=rs_ref.at[k - 1],
                send_sem=ssend_sems.at[k - 1],
                recv_sem=srecv_sems.at[k - 1],
                device_id=(p,),
                device_id_type=pl.DeviceIdType.MESH,
            )
            data.wait_send()
            sc.wait_send()

    return pl.pallas_call(
        body,
        out_shape=jax.ShapeDtypeStruct((m_out, n), jnp.float32),
        in_specs=[
            pl.BlockSpec(memory_space=pltpu.VMEM),
            pl.BlockSpec(memory_space=pltpu.VMEM),
        ],
        out_specs=pl.BlockSpec(memory_space=pltpu.VMEM),
        scratch_shapes=[
            pltpu.VMEM((N_DEV - 1, m_out, n), jnp.int8),
            pltpu.VMEM((N_DEV - 1, 8, 128), jnp.float32),
            pltpu.VMEM((N_DEV - 1, m_out, n), jnp.int8),
            pltpu.VMEM((N_DEV - 1, 8, 128), jnp.float32),
            pltpu.SemaphoreType.DMA((N_DEV - 1,)),
            pltpu.SemaphoreType.DMA((N_DEV - 1,)),
            pltpu.SemaphoreType.DMA((N_DEV - 1,)),
            pltpu.SemaphoreType.DMA((N_DEV - 1,)),
        ],
        compiler_params=pltpu.CompilerParams(collective_id=0),
    )(x, w_mat)
